# baseline (speedup 1.0000x reference)
import functools

import jax
import jax.numpy as jnp
from jax import lax
from jax.experimental import pallas as pl
from jax.experimental.pallas import tpu as pltpu
from jax.experimental.pallas import tpu_sc as plsc

N = 10000
E = 320000
DIN = 128
DH = 16
DOUT = 128

NC = 2
NS = 16
L = 16

NPAD = 10240
RPT = NPAD // NS
CH = 100
EPW = E // (NC * NS)
NCH = EPW // CH
RCH = 128

_mesh = plsc.VectorSubcoreMesh(core_axis_name="c", subcore_axis_name="s",
                               num_cores=NC, num_subcores=NS)
_sc_params = pltpu.CompilerParams(use_tc_tiling_on_sc=False)
_sc_params_nl = pltpu.CompilerParams(use_tc_tiling_on_sc=False,
                                     needs_layout_passes=False)


NHR = NPAD // L
HRT = NHR // NS
NRC = NHR // RCH


@functools.partial(
    pl.kernel,
    out_type=jax.ShapeDtypeStruct((NC, NHR, L), jnp.float32),
    mesh=_mesh,
    compiler_params=_sc_params_nl,
    scratch_types=[
        pltpu.VMEM_SHARED((NHR, L), jnp.float32),
        pltpu.VMEM((NHR, L), jnp.float32),
        pltpu.VMEM((2 * EPW,), jnp.int32),
        pltpu.VMEM((NRC, RCH), jnp.int32),
        pltpu.SemaphoreType.DMA,
    ],
)
def _deg_kernel(edges, iota, degs_out, hist_sh, hist_v, idx_v, iota_v, sem):
    c = lax.axis_index("c")
    s = lax.axis_index("s")
    hrows = pl.ds(s * HRT, HRT)
    ones = jnp.full((L,), 1.0, jnp.float32)
    zeros = jnp.zeros((L,), jnp.float32)
    lomask = jnp.full((L,), L - 1, jnp.int32)

    @pl.loop(0, NHR)
    def _(r):
        hist_v[r, :] = zeros

    pltpu.sync_copy(hist_v.at[hrows], hist_sh.at[hrows])

    pltpu.sync_copy(edges.at[c].at[2 * s], idx_v.at[pl.ds(0, EPW)])
    pltpu.sync_copy(edges.at[c].at[2 * s + 1], idx_v.at[pl.ds(EPW, EPW)])
    pltpu.sync_copy(iota, iota_v)

    @pl.loop(0, 2 * EPW // (8 * L))
    def _(j):
        for k in range(8):
            idx = idx_v[pl.ds(j * 8 * L + k * L, L)]
            plsc.addupdate_scatter(hist_v, [idx >> 4, idx & lomask], ones)

    plsc.subcore_barrier()

    @pl.loop(0, NRC)
    def _(j):
        pltpu.async_copy(hist_v.at[pl.ds(j * RCH, RCH)],
                         hist_sh.at[iota_v.at[j]], sem, add=True)

    @pl.loop(0, NRC)
    def _(j):
        pltpu.make_async_copy(hist_v.at[pl.ds(0, RCH)],
                              hist_sh.at[iota_v.at[0]], sem).wait()

    plsc.subcore_barrier()

    pltpu.sync_copy(hist_sh.at[hrows], degs_out.at[c].at[hrows])


def _agg_phase(table_sh, agg_sh, edges, srcv, dstv, r0, r1,
               s0, s1, w, buf_v, rows):
    @pl.loop(0, RPT)
    def _(r):
        buf_v[r, :] = jnp.zeros((L,), jnp.float32)

    pltpu.sync_copy(buf_v, agg_sh.at[rows])

    pltpu.sync_copy(edges.at[0].at[w], srcv)
    pltpu.sync_copy(edges.at[1].at[w], dstv)
    plsc.subcore_barrier()

    pltpu.async_copy(table_sh.at[srcv.at[0]], r0, s0)

    @pl.loop(0, NCH // 2)
    def _(jj):
        j0 = 2 * jj
        pltpu.make_async_copy(table_sh.at[srcv.at[j0]], r0, s0).wait()
        pltpu.async_copy(table_sh.at[srcv.at[j0 + 1]], r1, s1)
        pltpu.sync_copy(r0, agg_sh.at[dstv.at[j0]], add=True)
        pltpu.make_async_copy(table_sh.at[srcv.at[j0 + 1]], r1, s1).wait()

        @pl.when(jj + 1 < NCH // 2)
        def _():
            pltpu.async_copy(table_sh.at[srcv.at[j0 + 2]], r0, s0)

        pltpu.sync_copy(r1, agg_sh.at[dstv.at[j0 + 1]], add=True)

    plsc.subcore_barrier()


_agg_scratch = [
    pltpu.VMEM_SHARED((NPAD, L), jnp.float32),
    pltpu.VMEM_SHARED((NPAD, L), jnp.float32),
    pltpu.VMEM((RPT, L), jnp.float32),
    pltpu.VMEM((NCH, CH), jnp.int32),
    pltpu.VMEM((NCH, CH), jnp.int32),
    pltpu.VMEM((CH, L), jnp.float32),
    pltpu.VMEM((CH, L), jnp.float32),
    pltpu.SemaphoreType.DMA,
    pltpu.SemaphoreType.DMA,
]


@functools.partial(
    pl.kernel,
    out_type=jax.ShapeDtypeStruct((NC, NPAD, L), jnp.float32),
    mesh=_mesh,
    compiler_params=_sc_params_nl,
    scratch_types=_agg_scratch + [pltpu.VMEM((RPT,), jnp.float32)],
)
def _layer1_kernel(xw, norms, edges, part_out, table_sh, agg_sh, buf_v,
                   srcv, dstv, r0, r1, s0, s1, ns_v):
    c = lax.axis_index("c")
    s = lax.axis_index("s")
    w = c * NS + s
    rows = pl.ds(s * RPT, RPT)
    pltpu.sync_copy(xw.at[rows], buf_v)
    pltpu.sync_copy(norms.at[0].at[rows], ns_v)

    @pl.loop(0, RPT)
    def _(r):
        ns = plsc.load_gather(ns_v, [jnp.full((L,), r, jnp.int32)])
        buf_v[r, :] = buf_v[r, :] * ns

    pltpu.sync_copy(buf_v, table_sh.at[rows])
    _agg_phase(table_sh, agg_sh, edges, srcv, dstv, r0, r1, s0, s1,
               w, buf_v, rows)
    pltpu.sync_copy(agg_sh.at[rows], buf_v)
    pltpu.sync_copy(buf_v, part_out.at[c].at[rows])


@functools.partial(
    pl.kernel,
    out_type=jax.ShapeDtypeStruct((NC, NPAD, L), jnp.float32),
    mesh=_mesh,
    compiler_params=_sc_params_nl,
    scratch_types=_agg_scratch + [
        pltpu.VMEM((RPT, L), jnp.float32),
        pltpu.VMEM((RPT,), jnp.float32),
        pltpu.VMEM((RPT,), jnp.float32),
        pltpu.VMEM((L,), jnp.float32),
    ],
)
def _layer2_kernel(p, norms, b1, edges, part_out,
                   table_sh, agg_sh, a_v, srcv, dstv, r0, r1, s0, s1,
                   b_v, nd_v, ns_v, b1_v):
    c = lax.axis_index("c")
    s = lax.axis_index("s")
    w = c * NS + s
    rows = pl.ds(s * RPT, RPT)

    pltpu.sync_copy(b1, b1_v)
    bias = b1_v[...]

    pltpu.sync_copy(p.at[0].at[rows], a_v)
    pltpu.sync_copy(p.at[1].at[rows], b_v)
    pltpu.sync_copy(norms.at[1].at[rows], nd_v)
    pltpu.sync_copy(norms.at[0].at[rows], ns_v)

    @pl.loop(0, RPT)
    def _(r):
        ridx = jnp.full((L,), r, jnp.int32)
        nd = plsc.load_gather(nd_v, [ridx])
        ns = plsc.load_gather(ns_v, [ridx])
        h = (a_v[r, :] + b_v[r, :]) * nd + bias
        a_v[r, :] = jnp.maximum(h, 0.0) * ns

    pltpu.sync_copy(a_v, table_sh.at[rows])

    _agg_phase(table_sh, agg_sh, edges, srcv, dstv, r0, r1, s0, s1,
               w, a_v, rows)

    pltpu.sync_copy(agg_sh.at[rows], a_v)

    @pl.loop(0, RPT)
    def _(r):
        nd = plsc.load_gather(nd_v, [jnp.full((L,), r, jnp.int32)])
        a_v[r, :] = a_v[r, :] * nd

    pltpu.sync_copy(a_v, part_out.at[c].at[rows])


_RB1 = 1024
_RB2 = 1000


def _mm1a_body(x_ref, w_ref, xw_ref):
    xw_ref[...] = jnp.dot(x_ref[...], w_ref[...],
                          preferred_element_type=jnp.float32)


def _mm1a(x, W1):
    return pl.pallas_call(
        _mm1a_body,
        grid=(N // _RB2,),
        in_specs=[
            pl.BlockSpec((_RB2, DIN), lambda i: (i, 0)),
            pl.BlockSpec((DIN, DH), lambda i: (0, 0)),
        ],
        out_specs=pl.BlockSpec((_RB2, DH), lambda i: (i, 0)),
        out_shape=jax.ShapeDtypeStruct((NPAD, DH), jnp.float32),
    )(x, W1)


def _normk_body(d_ref, nrm_ref):
    nrm_ref[...] = jnp.where(d_ref[...] > 0.0, lax.rsqrt(d_ref[...]), 1.0)


def _normk(degs):
    return pl.pallas_call(
        _normk_body,
        grid=(NPAD // 2048,),
        in_specs=[pl.BlockSpec((2, 2048), lambda i: (0, i))],
        out_specs=pl.BlockSpec((2, 2048), lambda i: (0, i)),
        out_shape=jax.ShapeDtypeStruct((NC, NPAD), jnp.float32),
    )(degs)


def _mm2_body(a_ref, b_ref, w_ref, bias_ref, o_ref):
    h = a_ref[0] + b_ref[0]
    o_ref[...] = jnp.dot(h, w_ref[...],
                         preferred_element_type=jnp.float32) + bias_ref[...]


def _mm2(p2, W2, b2):
    return pl.pallas_call(
        _mm2_body,
        grid=(N // _RB2,),
        in_specs=[
            pl.BlockSpec((1, _RB2, DH), lambda i: (0, i, 0)),
            pl.BlockSpec((1, _RB2, DH), lambda i: (1, i, 0)),
            pl.BlockSpec((DH, DOUT), lambda i: (0, 0)),
            pl.BlockSpec((1, DOUT), lambda i: (0, 0)),
        ],
        out_specs=pl.BlockSpec((_RB2, DOUT), lambda i: (i, 0)),
        out_shape=jax.ShapeDtypeStruct((N, DOUT), jnp.float32),
    )(p2, p2, W2, b2.reshape(1, DOUT))


@jax.jit
def kernel(x, edge_index, W1, b1, W2, b2):
    edges = edge_index.reshape(2, NC * NS, NCH, CH)
    edges2 = edge_index.reshape(2, NC * NS, EPW)
    iota = jnp.arange(NHR, dtype=jnp.int32).reshape(NRC, RCH)

    degs = _deg_kernel(edges2, iota).reshape(NC, NPAD)
    xw = _mm1a(x, W1)
    norms = _normk(degs)
    p1 = _layer1_kernel(xw, norms, edges)
    p2 = _layer2_kernel(p1, norms, b1, edges)
    return _mm2(p2, W2, b2)


# speedup vs baseline: 1.0579x; 1.0579x over previous
import functools

import jax
import jax.numpy as jnp
from jax import lax
from jax.experimental import pallas as pl
from jax.experimental.pallas import tpu as pltpu
from jax.experimental.pallas import tpu_sc as plsc

N = 10000
E = 320000
DIN = 128
DH = 16
DOUT = 128

NC = 2
NS = 16
L = 16

NPAD = 10240
RPT = NPAD // NS
EPAD = 327680
CH = 128
EPW = EPAD // (NC * NS)
NCH = EPW // CH
RCH = 128

_mesh = plsc.VectorSubcoreMesh(core_axis_name="c", subcore_axis_name="s",
                               num_cores=NC, num_subcores=NS)
_sc_params = pltpu.CompilerParams(use_tc_tiling_on_sc=False)
_sc_params_nl = pltpu.CompilerParams(use_tc_tiling_on_sc=False,
                                     needs_layout_passes=False)


NHR = NPAD // L
HRT = NHR // NS
NRC = NHR // RCH


@functools.partial(
    pl.kernel,
    out_type=jax.ShapeDtypeStruct((NC, NHR, L), jnp.float32),
    mesh=_mesh,
    compiler_params=_sc_params_nl,
    scratch_types=[
        pltpu.VMEM_SHARED((NHR, L), jnp.float32),
        pltpu.VMEM((NHR, L), jnp.float32),
        pltpu.VMEM((2 * NCH, CH), jnp.int32),
        pltpu.VMEM((NRC, RCH), jnp.int32),
        pltpu.SemaphoreType.DMA,
    ],
)
def _deg_kernel(edges, iota, degs_out, hist_sh, hist_v, idx_v, iota_v, sem):
    c = lax.axis_index("c")
    s = lax.axis_index("s")
    hrows = pl.ds(s * HRT, HRT)
    ones = jnp.full((L,), 1.0, jnp.float32)
    zeros = jnp.zeros((L,), jnp.float32)
    lomask = jnp.full((L,), L - 1, jnp.int32)

    @pl.loop(0, NHR)
    def _(r):
        hist_v[r, :] = zeros

    pltpu.sync_copy(hist_v.at[hrows], hist_sh.at[hrows])

    pltpu.sync_copy(edges.at[c].at[2 * s], idx_v.at[pl.ds(0, NCH)])
    pltpu.sync_copy(edges.at[c].at[2 * s + 1], idx_v.at[pl.ds(NCH, NCH)])
    pltpu.sync_copy(iota, iota_v)

    @pl.loop(0, 2 * NCH)
    def _(j):
        for k in range(CH // L):
            idx = idx_v[j, pl.ds(k * L, L)]
            plsc.addupdate_scatter(hist_v, [idx >> 4, idx & lomask], ones)

    plsc.subcore_barrier()

    @pl.loop(0, NRC)
    def _(j):
        pltpu.async_copy(hist_v.at[pl.ds(j * RCH, RCH)],
                         hist_sh.at[iota_v.at[j]], sem, add=True)

    @pl.loop(0, NRC)
    def _(j):
        pltpu.make_async_copy(hist_v.at[pl.ds(0, RCH)],
                              hist_sh.at[iota_v.at[0]], sem).wait()

    plsc.subcore_barrier()

    pltpu.sync_copy(hist_sh.at[hrows], degs_out.at[c].at[hrows])


def _agg_phase(table_sh, agg_sh, edges, srcv, dstv, r0, r1,
               s0, s1, w, buf_v, rows):
    @pl.loop(0, RPT)
    def _(r):
        buf_v[r, :] = jnp.zeros((L,), jnp.float32)

    pltpu.sync_copy(buf_v, agg_sh.at[rows])

    pltpu.sync_copy(edges.at[0].at[w], srcv)
    pltpu.sync_copy(edges.at[1].at[w], dstv)
    plsc.subcore_barrier()

    pltpu.async_copy(table_sh.at[srcv.at[0]], r0, s0)

    @pl.loop(0, NCH // 2)
    def _(jj):
        j0 = 2 * jj
        pltpu.make_async_copy(table_sh.at[srcv.at[j0]], r0, s0).wait()
        pltpu.async_copy(table_sh.at[srcv.at[j0 + 1]], r1, s1)
        pltpu.sync_copy(r0, agg_sh.at[dstv.at[j0]], add=True)
        pltpu.make_async_copy(table_sh.at[srcv.at[j0 + 1]], r1, s1).wait()

        @pl.when(jj + 1 < NCH // 2)
        def _():
            pltpu.async_copy(table_sh.at[srcv.at[j0 + 2]], r0, s0)

        pltpu.sync_copy(r1, agg_sh.at[dstv.at[j0 + 1]], add=True)

    plsc.subcore_barrier()


_agg_scratch = [
    pltpu.VMEM_SHARED((NPAD, L), jnp.float32),
    pltpu.VMEM_SHARED((NPAD, L), jnp.float32),
    pltpu.VMEM((RPT, L), jnp.float32),
    pltpu.VMEM((NCH, CH), jnp.int32),
    pltpu.VMEM((NCH, CH), jnp.int32),
    pltpu.VMEM((CH, L), jnp.float32),
    pltpu.VMEM((CH, L), jnp.float32),
    pltpu.SemaphoreType.DMA,
    pltpu.SemaphoreType.DMA,
]


@functools.partial(
    pl.kernel,
    out_type=jax.ShapeDtypeStruct((NC, NPAD, L), jnp.float32),
    mesh=_mesh,
    compiler_params=_sc_params_nl,
    scratch_types=_agg_scratch + [pltpu.VMEM((RPT,), jnp.float32)],
)
def _layer1_kernel(xw, norms, edges, part_out, table_sh, agg_sh, buf_v,
                   srcv, dstv, r0, r1, s0, s1, ns_v):
    c = lax.axis_index("c")
    s = lax.axis_index("s")
    w = c * NS + s
    rows = pl.ds(s * RPT, RPT)
    pltpu.sync_copy(xw.at[rows], buf_v)
    pltpu.sync_copy(norms.at[0].at[rows], ns_v)

    @pl.loop(0, RPT)
    def _(r):
        ns = plsc.load_gather(ns_v, [jnp.full((L,), r, jnp.int32)])
        buf_v[r, :] = buf_v[r, :] * ns

    pltpu.sync_copy(buf_v, table_sh.at[rows])
    _agg_phase(table_sh, agg_sh, edges, srcv, dstv, r0, r1, s0, s1,
               w, buf_v, rows)
    pltpu.sync_copy(agg_sh.at[rows], buf_v)
    pltpu.sync_copy(buf_v, part_out.at[c].at[rows])


@functools.partial(
    pl.kernel,
    out_type=jax.ShapeDtypeStruct((NC, NPAD, L), jnp.float32),
    mesh=_mesh,
    compiler_params=_sc_params_nl,
    scratch_types=_agg_scratch + [
        pltpu.VMEM((RPT, L), jnp.float32),
        pltpu.VMEM((RPT,), jnp.float32),
        pltpu.VMEM((RPT,), jnp.float32),
        pltpu.VMEM((L,), jnp.float32),
    ],
)
def _layer2_kernel(p, norms, b1, edges, part_out,
                   table_sh, agg_sh, a_v, srcv, dstv, r0, r1, s0, s1,
                   b_v, nd_v, ns_v, b1_v):
    c = lax.axis_index("c")
    s = lax.axis_index("s")
    w = c * NS + s
    rows = pl.ds(s * RPT, RPT)

    pltpu.sync_copy(b1, b1_v)
    bias = b1_v[...]

    pltpu.sync_copy(p.at[0].at[rows], a_v)
    pltpu.sync_copy(p.at[1].at[rows], b_v)
    pltpu.sync_copy(norms.at[1].at[rows], nd_v)
    pltpu.sync_copy(norms.at[0].at[rows], ns_v)

    @pl.loop(0, RPT)
    def _(r):
        ridx = jnp.full((L,), r, jnp.int32)
        nd = plsc.load_gather(nd_v, [ridx])
        ns = plsc.load_gather(ns_v, [ridx])
        h = (a_v[r, :] + b_v[r, :]) * nd + bias
        a_v[r, :] = jnp.maximum(h, 0.0) * ns

    pltpu.sync_copy(a_v, table_sh.at[rows])

    _agg_phase(table_sh, agg_sh, edges, srcv, dstv, r0, r1, s0, s1,
               w, a_v, rows)

    pltpu.sync_copy(agg_sh.at[rows], a_v)

    @pl.loop(0, RPT)
    def _(r):
        nd = plsc.load_gather(nd_v, [jnp.full((L,), r, jnp.int32)])
        a_v[r, :] = a_v[r, :] * nd

    pltpu.sync_copy(a_v, part_out.at[c].at[rows])


_RB1 = 1024
_RB2 = 1000


def _mm1a_body(x_ref, w_ref, xw_ref):
    xw_ref[...] = jnp.dot(x_ref[...], w_ref[...],
                          preferred_element_type=jnp.float32)


def _mm1a(x_pad, W1):
    return pl.pallas_call(
        _mm1a_body,
        grid=(NPAD // _RB1,),
        in_specs=[
            pl.BlockSpec((_RB1, DIN), lambda i: (i, 0)),
            pl.BlockSpec((DIN, DH), lambda i: (0, 0)),
        ],
        out_specs=pl.BlockSpec((_RB1, DH), lambda i: (i, 0)),
        out_shape=jax.ShapeDtypeStruct((NPAD, DH), jnp.float32),
    )(x_pad, W1)


def _normk_body(d_ref, nrm_ref):
    nrm_ref[...] = jnp.where(d_ref[...] > 0.0, lax.rsqrt(d_ref[...]), 1.0)


def _normk(degs):
    return pl.pallas_call(
        _normk_body,
        grid=(NPAD // 2048,),
        in_specs=[pl.BlockSpec((2, 2048), lambda i: (0, i))],
        out_specs=pl.BlockSpec((2, 2048), lambda i: (0, i)),
        out_shape=jax.ShapeDtypeStruct((NC, NPAD), jnp.float32),
    )(degs)


def _mm2_body(a_ref, b_ref, w_ref, bias_ref, o_ref):
    h = a_ref[0] + b_ref[0]
    o_ref[...] = jnp.dot(h, w_ref[...],
                         preferred_element_type=jnp.float32) + bias_ref[...]


def _mm2(p2, W2, b2):
    return pl.pallas_call(
        _mm2_body,
        grid=(N // _RB2,),
        in_specs=[
            pl.BlockSpec((1, _RB2, DH), lambda i: (0, i, 0)),
            pl.BlockSpec((1, _RB2, DH), lambda i: (1, i, 0)),
            pl.BlockSpec((DH, DOUT), lambda i: (0, 0)),
            pl.BlockSpec((1, DOUT), lambda i: (0, 0)),
        ],
        out_specs=pl.BlockSpec((_RB2, DOUT), lambda i: (i, 0)),
        out_shape=jax.ShapeDtypeStruct((N, DOUT), jnp.float32),
    )(p2, p2, W2, b2.reshape(1, DOUT))


@jax.jit
def kernel(x, edge_index, W1, b1, W2, b2):
    pad = (jnp.arange(EPAD - E, dtype=jnp.int32) % (NPAD - N)) + N
    edges = jnp.concatenate(
        [edge_index, jnp.stack([pad, pad])], axis=1).reshape(2, NC * NS,
                                                            NCH, CH)
    x_pad = jnp.pad(x, ((0, NPAD - N), (0, 0)))
    iota = jnp.arange(NHR, dtype=jnp.int32).reshape(NRC, RCH)

    degs = _deg_kernel(edges, iota).reshape(NC, NPAD)
    xw = _mm1a(x_pad, W1)
    norms = _normk(degs)
    p1 = _layer1_kernel(xw, norms, edges)
    p2 = _layer2_kernel(p1, norms, b1, edges)
    return _mm2(p2, W2, b2)
